# baseline (speedup 1.0000x reference)
import functools
import jax
import jax.numpy as jnp
from jax import lax
from jax.experimental import pallas as pl
from jax.experimental.pallas import tpu as pltpu
from jax.experimental.pallas import tpu_sc as plsc


def _make_sc_bcast(S, N, E):
    info = plsc.get_sparse_core_info()
    nw = info.num_cores * info.num_subcores
    rows_per_w = S // nw
    mesh = plsc.VectorSubcoreMesh(core_axis_name="c", subcore_axis_name="s")

    @functools.partial(
        pl.kernel,
        out_type=jax.ShapeDtypeStruct((S, N, E), jnp.float32),
        mesh=mesh,
        scratch_types=[
            pltpu.VMEM((rows_per_w, 2, E), jnp.float32),
            pltpu.SemaphoreType.DMA,
            pltpu.SemaphoreType.DMA,
        ],
    )
    def sc_bcast(table_hbm, out_hbm, buf, sem_r, sem_w):
        wid = lax.axis_index("s") * info.num_cores + lax.axis_index("c")
        r0 = wid * rows_per_w
        reads = [
            pltpu.async_copy(table_hbm.at[pl.ds(r0, rows_per_w)], buf.at[:, j], sem_r)
            for j in range(2)
        ]
        for r in reads:
            r.wait()
        writes = [
            pltpu.async_copy(
                buf, out_hbm.at[pl.ds(r0, rows_per_w), pl.ds(2 * j, 2)], sem_w
            )
            for j in range(N // 2)
        ]
        for w in writes:
            w.wait()

    return sc_bcast


def kernel(x, table):
    S, N = x.shape
    _, E = table.shape
    return _make_sc_bcast(S, N, E)(table)


# speedup vs baseline: 1.0834x; 1.0834x over previous
import functools
import jax
import jax.numpy as jnp
from jax import lax
from jax.experimental import pallas as pl
from jax.experimental.pallas import tpu as pltpu
from jax.experimental.pallas import tpu_sc as plsc


def _make_sc_bcast(S, N, E):
    info = plsc.get_sparse_core_info()
    nw = info.num_cores * info.num_subcores
    rows_per_w = S // nw
    mesh = plsc.VectorSubcoreMesh(core_axis_name="c", subcore_axis_name="s")

    @functools.partial(
        pl.kernel,
        out_type=jax.ShapeDtypeStruct((S, N, E), jnp.float32),
        mesh=mesh,
        scratch_types=[
            pltpu.VMEM((rows_per_w, E), jnp.float32),
            pltpu.SemaphoreType.DMA,
            pltpu.SemaphoreType.DMA,
        ],
    )
    def sc_bcast(table_hbm, out_hbm, buf, sem_r, sem_w):
        wid = lax.axis_index("s") * info.num_cores + lax.axis_index("c")
        r0 = wid * rows_per_w
        ch = rows_per_w // 2
        reads = [
            pltpu.async_copy(
                table_hbm.at[pl.ds(r0 + k * ch, ch)], buf.at[pl.ds(k * ch, ch)], sem_r
            )
            for k in range(2)
        ]
        writes = []
        for k in range(2):
            reads[k].wait()
            for n in range(N):
                writes.append(
                    pltpu.async_copy(
                        buf.at[pl.ds(k * ch, ch)],
                        out_hbm.at[pl.ds(r0 + k * ch, ch), n],
                        sem_w,
                    )
                )
        for w in writes:
            w.wait()

    return sc_bcast


def kernel(x, table):
    S, N = x.shape
    _, E = table.shape
    return _make_sc_bcast(S, N, E)(table)
